# baseline (speedup 1.0000x reference)
import functools

import jax
import jax.numpy as jnp
from jax import lax
from jax.experimental import pallas as pl
from jax.experimental.pallas import tpu as pltpu
from jax.experimental.pallas import tpu_sc as plsc

O, I, K = 4096, 512, 2
CBS, G = 256, 8
OUT_W = I * G
NW = 32
ROWS_PER_W = O // NW
BLK = 8
NBLK = ROWS_PER_W // BLK
QUADS = I * K // 16
TAB_W = 16


def _body(cb_hbm, codes_hbm, out_hbm, tab_v,
          codes_v0, codes_v1, out_v0, out_v1,
          sem_in0, sem_in1, sem_out0, sem_out1):
    wid = lax.axis_index("s") * 2 + lax.axis_index("c")
    blk0 = wid * NBLK
    pltpu.sync_copy(cb_hbm, tab_v)

    codes_bufs = (codes_v0, codes_v1)
    out_bufs = (out_v0, out_v1)
    sems_in = (sem_in0, sem_in1)
    sems_out = (sem_out0, sem_out1)

    lanes = lax.iota(jnp.int32, 16)
    col = lanes
    pat = (lanes >> 3) * 2
    cb1_off = (lanes & 1) * (CBS * TAB_W)

    IK = I * K

    def start_in(half, bidx):
        for n in range(BLK):
            pltpu.async_copy(
                codes_hbm.at[(blk0 + bidx) * BLK + n],
                codes_bufs[half].at[pl.ds(n * IK, IK)], sems_in[half])

    def wait_in(half, bidx):
        for n in range(BLK):
            pltpu.make_async_copy(
                codes_hbm.at[(blk0 + bidx) * BLK + n],
                codes_bufs[half].at[pl.ds(n * IK, IK)], sems_in[half]).wait()

    def start_out(half, bidx):
        for n in range(BLK):
            pltpu.async_copy(
                out_bufs[half].at[pl.ds(n * OUT_W, OUT_W)],
                out_hbm.at[(blk0 + bidx) * BLK + n], sems_out[half])

    def wait_out(half, bidx):
        for n in range(BLK):
            pltpu.make_async_copy(
                out_bufs[half].at[pl.ds(n * OUT_W, OUT_W)],
                out_hbm.at[(blk0 + bidx) * BLK + n], sems_out[half]).wait()

    def compute_block(half):
        codes_buf = codes_bufs[half]
        out_buf = out_bufs[half]

        @plsc.parallel_loop(0, BLK * QUADS, unroll=8)
        def quad(m):
            cvec = codes_buf[pl.ds(m * 16, 16)]
            cidx = (cvec << 4) + cb1_off
            for t in range(4):
                a0 = jnp.take_along_axis(
                    cidx, pat + 4 * t, axis=0, mode="promise_in_bounds")
                a1 = jnp.take_along_axis(
                    cidx, pat + (4 * t + 1), axis=0,
                    mode="promise_in_bounds")
                v0 = plsc.load_gather(tab_v, [a0 | col])
                v1 = plsc.load_gather(tab_v, [a1 | col])
                out_buf[pl.ds(m * 64 + t * 16, 16)] = v0 + v1

    NBLK2 = NBLK // 2
    start_in(0, 0)
    start_in(1, 1)

    def pair_body(p, carry):
        for half in range(2):
            bidx = 2 * p + half
            wait_in(half, bidx)

            @pl.when(p > 0)
            def _():
                wait_out(half, bidx - 2)

            compute_block(half)
            start_out(half, bidx)

            @pl.when(p + 1 < NBLK2)
            def _():
                start_in(half, bidx + 2)

        return carry

    lax.fori_loop(0, NBLK2, pair_body, 0)
    wait_out(0, NBLK - 2)
    wait_out(1, NBLK - 1)


def kernel(codes, codebooks):
    flat_cb = codebooks.reshape(K * CBS, G)
    flat_cb = jnp.concatenate([flat_cb, flat_cb], axis=1).reshape(-1)
    codes2d = codes.reshape(O, I * K)
    mesh = plsc.VectorSubcoreMesh(core_axis_name="c", subcore_axis_name="s")
    k = functools.partial(
        pl.kernel,
        mesh=mesh,
        out_type=jax.ShapeDtypeStruct((O, OUT_W), jnp.float32),
        scratch_types=[
            pltpu.VMEM((K * CBS * TAB_W,), jnp.float32),
            pltpu.VMEM((BLK * I * K,), jnp.int32),
            pltpu.VMEM((BLK * I * K,), jnp.int32),
            pltpu.VMEM((BLK * OUT_W,), jnp.float32),
            pltpu.VMEM((BLK * OUT_W,), jnp.float32),
            pltpu.SemaphoreType.DMA,
            pltpu.SemaphoreType.DMA,
            pltpu.SemaphoreType.DMA,
            pltpu.SemaphoreType.DMA,
        ],
        compiler_params=pltpu.CompilerParams(needs_layout_passes=False),
    )(_body)
    return k(flat_cb, codes2d)


# speedup vs baseline: 1.0076x; 1.0076x over previous
import functools

import jax
import jax.numpy as jnp
from jax import lax
from jax.experimental import pallas as pl
from jax.experimental.pallas import tpu as pltpu
from jax.experimental.pallas import tpu_sc as plsc

O, I, K = 4096, 512, 2
CBS, G = 256, 8
OUT_W = I * G
NW = 32
ROWS_PER_W = O // NW
BLK = 8
NBLK = ROWS_PER_W // BLK
QUADS = I * K // 16
TAB_W = 16


def _body(cb_hbm, codes_hbm, out_hbm, tab_v,
          codes_v0, codes_v1, out_v0, out_v1,
          sem_in0, sem_in1, sem_out0, sem_out1):
    wid = lax.axis_index("s") * 2 + lax.axis_index("c")
    blk0 = wid * NBLK
    pltpu.sync_copy(cb_hbm, tab_v)

    codes_bufs = (codes_v0, codes_v1)
    out_bufs = (out_v0, out_v1)
    sems_in = (sem_in0, sem_in1)
    sems_out = (sem_out0, sem_out1)

    lanes = lax.iota(jnp.int32, 16)
    col = lanes
    pat = (lanes >> 3) * 2
    cb1_off = (lanes & 1) * (CBS * TAB_W)

    IK = I * K

    def start_in(half, bidx):
        for n in range(BLK):
            pltpu.async_copy(
                codes_hbm.at[(blk0 + bidx) * BLK + n],
                codes_bufs[half].at[pl.ds(n * IK, IK)], sems_in[half])

    def wait_in(half, bidx):
        for n in range(BLK):
            pltpu.make_async_copy(
                codes_hbm.at[(blk0 + bidx) * BLK + n],
                codes_bufs[half].at[pl.ds(n * IK, IK)], sems_in[half]).wait()

    def start_out(half, bidx):
        for n in range(BLK):
            pltpu.async_copy(
                out_bufs[half].at[pl.ds(n * OUT_W, OUT_W)],
                out_hbm.at[(blk0 + bidx) * BLK + n], sems_out[half])

    def wait_out(half, bidx):
        for n in range(BLK):
            pltpu.make_async_copy(
                out_bufs[half].at[pl.ds(n * OUT_W, OUT_W)],
                out_hbm.at[(blk0 + bidx) * BLK + n], sems_out[half]).wait()

    def compute_block(half):
        codes_buf = codes_bufs[half]
        out_buf = out_bufs[half]

        @plsc.parallel_loop(0, BLK * QUADS, unroll=4)
        def quad(m):
            cvec = codes_buf[pl.ds(m * 16, 16)]
            cidx = (cvec << 4) + cb1_off
            for t in range(4):
                a0 = jnp.take_along_axis(
                    cidx, pat + 4 * t, axis=0, mode="promise_in_bounds")
                a1 = jnp.take_along_axis(
                    cidx, pat + (4 * t + 1), axis=0,
                    mode="promise_in_bounds")
                v0 = plsc.load_gather(tab_v, [a0 | col])
                v1 = plsc.load_gather(tab_v, [a1 | col])
                out_buf[pl.ds(m * 64 + t * 16, 16)] = v0 + v1

    NBLK2 = NBLK // 2
    start_in(0, 0)
    start_in(1, 1)

    def pair_body(p, carry):
        for half in range(2):
            bidx = 2 * p + half
            wait_in(half, bidx)

            @pl.when(p > 0)
            def _():
                wait_out(half, bidx - 2)

            compute_block(half)
            start_out(half, bidx)

            @pl.when(p + 1 < NBLK2)
            def _():
                start_in(half, bidx + 2)

        return carry

    lax.fori_loop(0, NBLK2, pair_body, 0)
    wait_out(0, NBLK - 2)
    wait_out(1, NBLK - 1)


def kernel(codes, codebooks):
    flat_cb = codebooks.reshape(K * CBS, G)
    flat_cb = jnp.concatenate([flat_cb, flat_cb], axis=1).reshape(-1)
    codes2d = codes.reshape(O, I * K)
    mesh = plsc.VectorSubcoreMesh(core_axis_name="c", subcore_axis_name="s")
    k = functools.partial(
        pl.kernel,
        mesh=mesh,
        out_type=jax.ShapeDtypeStruct((O, OUT_W), jnp.float32),
        scratch_types=[
            pltpu.VMEM((K * CBS * TAB_W,), jnp.float32),
            pltpu.VMEM((BLK * I * K,), jnp.int32),
            pltpu.VMEM((BLK * I * K,), jnp.int32),
            pltpu.VMEM((BLK * OUT_W,), jnp.float32),
            pltpu.VMEM((BLK * OUT_W,), jnp.float32),
            pltpu.SemaphoreType.DMA,
            pltpu.SemaphoreType.DMA,
            pltpu.SemaphoreType.DMA,
            pltpu.SemaphoreType.DMA,
        ],
        compiler_params=pltpu.CompilerParams(needs_layout_passes=False),
    )(_body)
    return k(flat_cb, codes2d)


# speedup vs baseline: 1.0933x; 1.0851x over previous
import functools

import jax
import jax.numpy as jnp
from jax import lax
from jax.experimental import pallas as pl
from jax.experimental.pallas import tpu as pltpu
from jax.experimental.pallas import tpu_sc as plsc

O, I, K = 4096, 512, 2
CBS, G = 256, 8
OUT_W = I * G
NW = 32
ROWS_PER_W = O // NW
BLK = 8
NBLK = ROWS_PER_W // BLK
TAB_W = 16
OCTS = I * K // 32
SROWS = I * K // 128


def _body(cb_hbm, codes_hbm, out_hbm, tab_v,
          codes_v0, codes_v1, out_v0, out_v1,
          sem_in0, sem_in1, sem_out0, sem_out1):
    wid = lax.axis_index("s") * 2 + lax.axis_index("c")
    blk0 = wid * NBLK
    pltpu.sync_copy(cb_hbm, tab_v)

    codes_bufs = (codes_v0, codes_v1)
    out_bufs = (out_v0, out_v1)
    sems_in = (sem_in0, sem_in1)
    sems_out = (sem_out0, sem_out1)

    lanes = lax.iota(jnp.int32, 16)
    col = lanes
    pat2 = lanes >> 3

    def start_in(half, bidx):
        return pltpu.async_copy(
            codes_hbm.at[pl.ds((blk0 + bidx) * BLK, BLK)],
            codes_bufs[half], sems_in[half])

    def wait_in(half, bidx):
        pltpu.make_async_copy(
            codes_hbm.at[pl.ds((blk0 + bidx) * BLK, BLK)],
            codes_bufs[half], sems_in[half]).wait()

    def start_out(half, bidx):
        for n in range(BLK):
            pltpu.async_copy(
                out_bufs[half].at[pl.ds(n * OUT_W, OUT_W)],
                out_hbm.at[(blk0 + bidx) * BLK + n], sems_out[half])

    def wait_out(half, bidx):
        for n in range(BLK):
            pltpu.make_async_copy(
                out_bufs[half].at[pl.ds(n * OUT_W, OUT_W)],
                out_hbm.at[(blk0 + bidx) * BLK + n], sems_out[half]).wait()

    def compute_block(half):
        codes_buf = codes_bufs[half]
        out_buf = out_bufs[half]
        for n in range(BLK):
            @plsc.parallel_loop(0, OCTS, unroll=4)
            def octet(m):
                t = m >> 3
                b = m & 7
                cv0 = codes_buf[n, 2 * t, pl.ds(b * 16, 16)]
                cv1 = codes_buf[n, 2 * t + 1, pl.ds(b * 16, 16)]
                ci0 = cv0 << 4
                ci1 = (cv1 << 4) + (CBS * TAB_W)
                base = n * OUT_W + m * 128
                for j in range(8):
                    a0 = jnp.take_along_axis(
                        ci0, pat2 + 2 * j, axis=0, mode="promise_in_bounds")
                    a1 = jnp.take_along_axis(
                        ci1, pat2 + 2 * j, axis=0, mode="promise_in_bounds")
                    v0 = plsc.load_gather(tab_v, [a0 | col])
                    v1 = plsc.load_gather(tab_v, [a1 | col])
                    out_buf[pl.ds(base + j * 16, 16)] = v0 + v1

    NBLK2 = NBLK // 2
    start_in(0, 0)
    start_in(1, 1)

    def pair_body(p, carry):
        for half in range(2):
            bidx = 2 * p + half
            wait_in(half, bidx)

            @pl.when(p > 0)
            def _():
                wait_out(half, bidx - 2)

            compute_block(half)
            start_out(half, bidx)

            @pl.when(p + 1 < NBLK2)
            def _():
                start_in(half, bidx + 2)

        return carry

    lax.fori_loop(0, NBLK2, pair_body, 0)
    wait_out(0, NBLK - 2)
    wait_out(1, NBLK - 1)


def kernel(codes, codebooks):
    flat_cb = codebooks.reshape(K * CBS, G)
    flat_cb = jnp.concatenate([flat_cb, flat_cb], axis=1).reshape(-1)
    codes_t = codes.reshape(O, 4, 128, K).transpose(0, 1, 3, 2)
    codes_t = codes_t.reshape(O, SROWS, 128)
    mesh = plsc.VectorSubcoreMesh(core_axis_name="c", subcore_axis_name="s")
    k = functools.partial(
        pl.kernel,
        mesh=mesh,
        out_type=jax.ShapeDtypeStruct((O, OUT_W), jnp.float32),
        scratch_types=[
            pltpu.VMEM((K * CBS * TAB_W,), jnp.float32),
            pltpu.VMEM((BLK, SROWS, 128), jnp.int32),
            pltpu.VMEM((BLK, SROWS, 128), jnp.int32),
            pltpu.VMEM((BLK * OUT_W,), jnp.float32),
            pltpu.VMEM((BLK * OUT_W,), jnp.float32),
            pltpu.SemaphoreType.DMA,
            pltpu.SemaphoreType.DMA,
            pltpu.SemaphoreType.DMA,
            pltpu.SemaphoreType.DMA,
        ],
        compiler_params=pltpu.CompilerParams(needs_layout_passes=False),
    )(_body)
    return k(flat_cb, codes_t)
